# Initial kernel scaffold; baseline (speedup 1.0000x reference)
import jax
import jax.numpy as jnp
from jax.experimental import pallas as pl


def kernel(x, edge_index, W1, b1, W2, b2):
    raise NotImplementedError("write your pallas kernel here")


# speedup vs baseline: 61.2437x; 61.2437x over previous
import functools

import jax
import jax.numpy as jnp
from jax import lax
from jax.experimental import pallas as pl
from jax.experimental.pallas import tpu as pltpu
from jax.experimental.pallas import tpu_sc as plsc

NC = 2
NS = 16
NW = NC * NS
LANE = 128
K = 8


def _node_pad(n):
    return ((n + 1 + 2047) // 2048) * 2048


def _make_edge_pass(ncols, gather, rows, np_):
    zb = np_ // NS
    rpw = rows // NW
    nblk = rpw // K

    mesh = plsc.VectorSubcoreMesh(
        core_axis_name="c", subcore_axis_name="s", num_cores=NC,
        num_subcores=NS)

    scratch = [pltpu.VMEM((K, LANE), jnp.int32)]
    if gather:
        scratch.append(pltpu.VMEM((K, LANE), jnp.int32))
        scratch += [pltpu.VMEM((K, LANE), jnp.float32) for _ in range(ncols)]
    else:
        scratch.append(pltpu.VMEM((LANE,), jnp.float32))
    scratch.append(pltpu.VMEM((zb,), jnp.float32))
    scratch += [pltpu.VMEM_SHARED((np_,), jnp.float32) for _ in range(ncols)]
    scratch += [pltpu.SemaphoreType.DMA] * 3

    def body(*refs):
        pos = 0
        dst_r = refs[pos]; pos += 1
        if gather:
            src_r = refs[pos]; pos += 1
            tables = refs[pos:pos + ncols]; pos += ncols
        out_r = refs[pos]; pos += 1
        dst_buf = refs[pos]; pos += 1
        if gather:
            src_buf = refs[pos]; pos += 1
            val_bufs = refs[pos:pos + ncols]; pos += ncols
        else:
            ones_buf = refs[pos]; pos += 1
        zbuf = refs[pos]; pos += 1
        accs = refs[pos:pos + ncols]; pos += ncols
        sem_i, sem_g, sem_s = refs[pos:pos + 3]

        cid = lax.axis_index("c")
        sid = lax.axis_index("s")
        wid = sid * NC + cid

        def zfill(i, _):
            zbuf[pl.ds(i * 16, 16)] = jnp.zeros((16,), jnp.float32)
            return 0
        lax.fori_loop(0, zb // 16, zfill, 0)
        if not gather:
            for j in range(LANE // 16):
                ones_buf[pl.ds(j * 16, 16)] = jnp.ones((16,), jnp.float32)
        for acc in accs:
            pltpu.sync_copy(zbuf, acc.at[pl.ds(sid * zb, zb)])
        plsc.subcore_barrier()

        base = wid * rpw

        def blk(b, _):
            row0 = base + b * K
            cps = [pltpu.async_copy(dst_r.at[pl.ds(row0, K)], dst_buf, sem_i)]
            if gather:
                cps.append(
                    pltpu.async_copy(src_r.at[pl.ds(row0, K)], src_buf, sem_i))
            for cp in cps:
                cp.wait()
            if gather:
                gs = []
                for t, vb in zip(tables, val_bufs):
                    for j in range(K):
                        gs.append(pltpu.async_copy(
                            t.at[src_buf.at[j]], vb.at[j], sem_g))
                for g in gs:
                    g.wait()
                srcs = val_bufs
            else:
                srcs = [None]
            ss = []
            for acc, vb in zip(accs, srcs):
                for j in range(K):
                    v = ones_buf if vb is None else vb.at[j]
                    ss.append(pltpu.async_copy(
                        v, acc.at[dst_buf.at[j]], sem_s, add=True))
            for s_ in ss:
                s_.wait()
            return 0
        lax.fori_loop(0, nblk, blk, 0)

        plsc.subcore_barrier()
        for ci, acc in enumerate(accs):
            off = (cid * ncols + ci) * np_ + sid * zb
            pltpu.sync_copy(acc.at[pl.ds(sid * zb, zb)],
                            out_r.at[pl.ds(off, zb)])

    return pl.kernel(
        body,
        out_type=jax.ShapeDtypeStruct((NC * ncols * np_,), jnp.float32),
        mesh=mesh,
        scratch_types=scratch,
    )


def _tc1_body(degp_ref, xt_ref, dinv_ref, g0_ref, g1_ref, g2_ref):
    deg = degp_ref[0, 0] + degp_ref[1, 0] + 1.0
    dinv = lax.rsqrt(deg)
    dinv_ref[...] = dinv
    g0_ref[...] = xt_ref[0] * dinv
    g1_ref[...] = xt_ref[1] * dinv
    g2_ref[...] = xt_ref[2] * dinv


def _tc2_body(accp_ref, g0_ref, g1_ref, g2_ref, dinv_ref, w1_ref, b1_ref,
              w2_ref, o0_ref, o1_ref):
    dinv = dinv_ref[...]
    gs = (g0_ref[...], g1_ref[...], g2_ref[...])
    a = [dinv * (accp_ref[0, c] + accp_ref[1, c] + gs[c]) for c in range(3)]
    hs = []
    for j in range(8):
        v = a[0] * w1_ref[0, j] + a[1] * w1_ref[1, j] + a[2] * w1_ref[2, j]
        hs.append(jnp.maximum(v + b1_ref[j], 0.0))
    for kk, oref in enumerate((o0_ref, o1_ref)):
        z = hs[0] * w2_ref[0, kk]
        for j in range(1, 8):
            z = z + hs[j] * w2_ref[j, kk]
        oref[...] = z * dinv


def _tc3_body(accp_ref, g0_ref, g1_ref, dinv_ref, b2_ref, out_ref):
    dinv = dinv_ref[...]
    gs = (g0_ref[...], g1_ref[...])
    for kk in range(2):
        out_ref[kk] = dinv * (accp_ref[0, kk] + accp_ref[1, kk] + gs[kk]) \
            + b2_ref[kk]


def _vspec():
    return pl.BlockSpec(memory_space=pltpu.VMEM)


def _sspec():
    return pl.BlockSpec(memory_space=pltpu.SMEM)


def kernel(x, edge_index, W1, b1, W2, b2):
    n = x.shape[0]
    e = edge_index.shape[1]
    np_ = _node_pad(n)
    nr = np_ // 128
    unit = NW * K * LANE
    e_pad = ((e + unit - 1) // unit) * unit
    rows = e_pad // LANE

    src = edge_index[0]
    dst = edge_index[1]
    pad = e_pad - e
    srcp = jnp.concatenate(
        [src, jnp.zeros((pad,), jnp.int32)]).reshape(rows, LANE)
    dstp = jnp.concatenate(
        [dst, jnp.full((pad,), n, jnp.int32)]).reshape(rows, LANE)
    xt = jnp.concatenate(
        [x, jnp.zeros((np_ - n, 3), x.dtype)]).T.reshape(3, nr, 128)

    deg_pass = _make_edge_pass(1, False, rows, np_)
    agg3_pass = _make_edge_pass(3, True, rows, np_)
    agg2_pass = _make_edge_pass(2, True, rows, np_)

    degp = deg_pass(dstp)

    shp = jax.ShapeDtypeStruct((nr, 128), jnp.float32)
    dinv, g10, g11, g12 = pl.pallas_call(
        _tc1_body,
        out_shape=[shp, shp, shp, shp],
        in_specs=[_vspec(), _vspec()],
        out_specs=[_vspec()] * 4,
    )(degp.reshape(NC, 1, nr, 128), xt)

    acc1 = agg3_pass(dstp, srcp, g10.reshape(np_), g11.reshape(np_),
                     g12.reshape(np_))

    g20, g21 = pl.pallas_call(
        _tc2_body,
        out_shape=[shp, shp],
        in_specs=[_vspec()] * 5 + [_sspec()] * 3,
        out_specs=[_vspec()] * 2,
    )(acc1.reshape(NC, 3, nr, 128), g10, g11, g12, dinv, W1, b1, W2)

    acc2 = agg2_pass(dstp, srcp, g20.reshape(np_), g21.reshape(np_))

    outp = pl.pallas_call(
        _tc3_body,
        out_shape=jax.ShapeDtypeStruct((2, nr, 128), jnp.float32),
        in_specs=[_vspec()] * 4 + [_sspec()],
        out_specs=_vspec(),
    )(acc2.reshape(NC, 2, nr, 128), g20, g21, dinv, b2)

    return outp.reshape(2, np_)[:, :n].T
